# baseline (speedup 1.0000x reference)
import functools

import jax
import jax.numpy as jnp
from jax import lax
from jax.experimental import pallas as pl
from jax.experimental.pallas import tpu as pltpu
from jax.experimental.pallas import tpu_sc as plsc

B = 16384
D = 16
R = 5
NC = 2
NS = 16
NW = NC * NS
BSC = 2048
BTC = B - BSC
SC_OFF = BTC
CHUNK = BSC // NW
GROUP = 16
PAIR = 2 * GROUP
TILE = BTC // 2


def _sc_body(gut_hbm, git_hbm, psp_hbm, asp_hbm, rsp_hbm,
             xui_hbm, puit_hbm,
             gut_v, git_v, puit_v, xui_v, psp_v, asp_v, rsp_v):
    wid = lax.axis_index("s") * NC + lax.axis_index("c")
    sbase = wid * CHUNK

    pltpu.sync_copy(gut_hbm.at[:, pl.ds(SC_OFF + sbase, CHUNK)], gut_v)
    pltpu.sync_copy(git_hbm.at[:, pl.ds(SC_OFF + sbase, CHUNK)], git_v)
    pltpu.sync_copy(psp_hbm, psp_v)
    pltpu.sync_copy(asp_hbm, asp_v)
    pltpu.sync_copy(rsp_hbm, rsp_v)

    a_vec = [[asp_v[r, s, :] for s in range(2)] for r in range(R)]
    r_vec = [rsp_v[r, :] for r in range(R)]

    def finish_group(off, m0, m1):
        p_rel = [a_vec[r][0] * m0 + a_vec[r][1] * m1 for r in range(R)]
        mx = jnp.maximum(jnp.maximum(jnp.maximum(p_rel[0], p_rel[1]),
                                     jnp.maximum(p_rel[2], p_rel[3])),
                         p_rel[4])
        ex = [jnp.exp(p_rel[r] - mx) for r in range(R)]
        den = ((ex[0] + ex[1]) + (ex[2] + ex[3])) + ex[4]
        num = ((r_vec[0] * ex[0] + r_vec[1] * ex[1])
               + (r_vec[2] * ex[2] + r_vec[3] * ex[3])) + r_vec[4] * ex[4]
        xui_v[pl.ds(off, GROUP)] = num / den
        for r in range(R):
            puit_v[r, pl.ds(off, GROUP)] = p_rel[r]

    def pair_body(g, carry):
        del carry
        off_a = g * PAIR
        off_b = off_a + GROUP
        zu_a = [gut_v[i, pl.ds(off_a, GROUP)] for i in range(D)]
        zu_b = [gut_v[i, pl.ds(off_b, GROUP)] for i in range(D)]
        ma0 = mb0 = ma1 = mb1 = None
        for j in range(D):
            zi_aj = git_v[j, pl.ds(off_a, GROUP)]
            zi_bj = git_v[j, pl.ds(off_b, GROUP)]
            ca0 = cb0 = ca1 = cb1 = None
            for i in range(D):
                p0 = psp_v[0, i, j, :]
                p1 = psp_v[1, i, j, :]
                ca0 = p0 * zu_a[i] if ca0 is None else ca0 + p0 * zu_a[i]
                cb0 = p0 * zu_b[i] if cb0 is None else cb0 + p0 * zu_b[i]
                ca1 = p1 * zu_a[i] if ca1 is None else ca1 + p1 * zu_a[i]
                cb1 = p1 * zu_b[i] if cb1 is None else cb1 + p1 * zu_b[i]
            ma0 = zi_aj * ca0 if ma0 is None else ma0 + zi_aj * ca0
            mb0 = zi_bj * cb0 if mb0 is None else mb0 + zi_bj * cb0
            ma1 = zi_aj * ca1 if ma1 is None else ma1 + zi_aj * ca1
            mb1 = zi_bj * cb1 if mb1 is None else mb1 + zi_bj * cb1
        finish_group(off_a, ma0, ma1)
        finish_group(off_b, mb0, mb1)
        return 0

    lax.fori_loop(0, CHUNK // PAIR, pair_body, 0)

    pltpu.sync_copy(xui_v, xui_hbm.at[pl.ds(sbase, CHUNK)])
    pltpu.sync_copy(puit_v, puit_hbm.at[:, pl.ds(sbase, CHUNK)])


def _sc_call(gut, git, psp, asp, rsp):
    mesh = plsc.VectorSubcoreMesh(core_axis_name="c", subcore_axis_name="s")
    fn = pl.kernel(
        _sc_body,
        mesh=mesh,
        out_type=(
            jax.ShapeDtypeStruct((BSC,), jnp.float32),
            jax.ShapeDtypeStruct((R, BSC), jnp.float32),
        ),
        compiler_params=pltpu.CompilerParams(
            needs_layout_passes=False, use_tc_tiling_on_sc=False),
        scratch_types=[
            pltpu.VMEM((D, CHUNK), jnp.float32),
            pltpu.VMEM((D, CHUNK), jnp.float32),
            pltpu.VMEM((R, CHUNK), jnp.float32),
            pltpu.VMEM((CHUNK,), jnp.float32),
            pltpu.VMEM((2, D, D, D), jnp.float32),
            pltpu.VMEM((R, 2, D), jnp.float32),
            pltpu.VMEM((R, D), jnp.float32),
        ],
    )
    return fn(gut, git, psp, asp, rsp)


def _tc_body(gut_ref, git_ref, pt_ref, apad_ref, relcol_ref, xui_ref, pui_ref):
    gut = gut_ref[...]
    git = git_ref[...]
    t0 = jnp.dot(pt_ref[0], gut, preferred_element_type=jnp.float32)
    t1 = jnp.dot(pt_ref[1], gut, preferred_element_type=jnp.float32)
    m0 = jnp.sum(t0 * git, axis=0, keepdims=True)
    m1 = jnp.sum(t1 * git, axis=0, keepdims=True)
    mstk = jnp.concatenate([m0, m1], axis=0)
    pstk = jnp.dot(apad_ref[...], mstk, preferred_element_type=jnp.float32)
    valid = lax.broadcasted_iota(jnp.int32, (8, 1), 0) < R
    neg_inf = jnp.float32(float("-inf"))
    mx = jnp.max(jnp.where(valid, pstk, neg_inf), axis=0, keepdims=True)
    ex = jnp.where(valid, jnp.exp(pstk - mx), 0.0)
    den = jnp.sum(ex, axis=0, keepdims=True)
    num = jnp.sum(relcol_ref[...] * ex, axis=0, keepdims=True)
    xui_ref[...] = num / den
    pui_ref[...] = pstk[:R]


def _tc_call(gut, git, pt, apad, relcol):
    grid = (BTC // TILE,)
    return pl.pallas_call(
        _tc_body,
        grid=grid,
        in_specs=[
            pl.BlockSpec((D, TILE), lambda b: (0, b)),
            pl.BlockSpec((D, TILE), lambda b: (0, b)),
            pl.BlockSpec((2, D, D), lambda b: (0, 0, 0)),
            pl.BlockSpec((8, 2), lambda b: (0, 0)),
            pl.BlockSpec((8, 1), lambda b: (0, 0)),
        ],
        out_specs=[
            pl.BlockSpec((1, TILE), lambda b: (0, b)),
            pl.BlockSpec((R, TILE), lambda b: (0, b)),
        ],
        out_shape=[
            jax.ShapeDtypeStruct((1, BTC), jnp.float32),
            jax.ShapeDtypeStruct((R, BTC), jnp.float32),
        ],
        compiler_params=pltpu.CompilerParams(
            dimension_semantics=("arbitrary",)),
    )(gut, git, pt, apad, relcol)


@jax.jit
def _hybrid(gu, gi, P, A, relations):
    gut = gu.T
    git = gi.T
    pt = jnp.swapaxes(P, 1, 2)
    apad = jnp.zeros((8, 2), jnp.float32).at[:R].set(A)
    relcol = jnp.zeros((8, 1), jnp.float32).at[:R, 0].set(relations)
    psp = jnp.broadcast_to(P[:, :, :, None], (2, D, D, D))
    asp = jnp.broadcast_to(A[:, :, None], (R, 2, D))
    rsp = jnp.broadcast_to(relations[:, None], (R, D))

    xui_sc, puit_sc = _sc_call(gut, git, psp, asp, rsp)
    xui_tc, puit_tc = _tc_call(gut, git, pt, apad, relcol)

    xui = jnp.concatenate([xui_tc.reshape(BTC), xui_sc], axis=0)
    puit = jnp.concatenate([puit_tc, puit_sc], axis=1)
    return (xui, puit.T)


def kernel(gu, gi, P, A, relations):
    return _hybrid(jnp.squeeze(gu), jnp.squeeze(gi), P, A, relations)
